# baseline (speedup 1.0000x reference)
import functools
import jax
import jax.numpy as jnp
from jax import lax
from jax.experimental import pallas as pl
from jax.experimental.pallas import tpu as pltpu
from jax.experimental.pallas import tpu_sc as plsc

N = 10000
E = 320000
DF = 128
DE = 16
H = 256
HH = 128
DEPTH = 4
G = 256

BE = 2560
_f32 = jnp.float32
_i32 = jnp.int32
PREC = lax.Precision.DEFAULT


def _dot(a, b, prec=None):
    return jnp.dot(a, b, preferred_element_type=_f32,
                   precision=PREC if prec is None else prec)


def _cat(ref):
    return jnp.concatenate([ref[0], ref[1]], axis=1)


def _init_body(xr_ref, ea_ref, w_ref, b_ref, h0_ref):
    xe = jnp.concatenate([xr_ref[...], ea_ref[...]], axis=1)
    h0_ref[...] = jax.nn.relu(_dot(xe, w_ref[...]) + b_ref[...])


def _pair_swap(g):
    up = jnp.concatenate([g[1:], g[:1]], axis=0)
    dn = jnp.concatenate([g[-1:], g[:-1]], axis=0)
    par = lax.broadcasted_iota(jnp.int32, (g.shape[0], 1), 0) % 2
    return jnp.where(par == 0, up, dn)


def _layer_body(gath_ref, h_ref, h0_ref, b_ref, w_ref, o_ref, *, split):
    t = gath_ref[...] - _pair_swap(h_ref[...])
    o = jax.nn.relu(_dot(t, w_ref[...]) + b_ref[...] + h0_ref[...])
    if split:
        o_ref[0] = o[:, :HH]
        o_ref[1] = o[:, HH:]
    else:
        o_ref[...] = o


def _final_body(x_ref, s_ref, wen_ref, ben_ref, batch_ref, wf1_ref,
                bf1_ref, wf2_ref, bf2_ref, o_ref):
    q = jnp.concatenate([x_ref[...], _cat(s_ref)], axis=1)
    hn = jax.nn.relu(_dot(q, wen_ref[...]) + ben_ref[...])
    gid = lax.broadcasted_iota(jnp.int32, (G, N), 0)
    oh = (batch_ref[...] == gid).astype(_f32)
    pooled = _dot(oh, hn, prec=lax.Precision.HIGHEST)
    f1 = jax.nn.relu(_dot(pooled, wf1_ref[...]) + bf1_ref[...])
    o_ref[...] = _dot(f1, wf2_ref[...]) + bf2_ref[...]


def _split_spec():
    return pl.BlockSpec((2, BE, HH), lambda i: (0, i, 0))


def _whole(shape):
    return pl.BlockSpec(shape, lambda *i: tuple(0 for _ in shape))


def _tc_init(xr, edge_attr, w_ei, b_ei):
    return pl.pallas_call(
        _init_body,
        grid=(E // BE,),
        in_specs=[
            pl.BlockSpec((BE, DF), lambda i: (i, 0)),
            pl.BlockSpec((BE, DE), lambda i: (i, 0)),
            _whole((DF + DE, H)),
            _whole((1, H)),
        ],
        out_specs=_edge_spec(),
        out_shape=jax.ShapeDtypeStruct((E, H), _f32),
    )(xr, edge_attr, w_ei, b_ei)


def _edge_spec():
    return pl.BlockSpec((BE, H), lambda i: (i, 0))


def _tc_layer(gath, h, h0, b, w, split=False):
    if split:
        ospec = _split_spec()
        oshape = jax.ShapeDtypeStruct((2, E, HH), _f32)
    else:
        ospec = _edge_spec()
        oshape = jax.ShapeDtypeStruct((E, H), _f32)
    return pl.pallas_call(
        functools.partial(_layer_body, split=split),
        grid=(E // BE,),
        in_specs=[
            _edge_spec(),
            _edge_spec(),
            _edge_spec(),
            _whole((1, H)),
            _whole((H, H)),
        ],
        out_specs=ospec,
        out_shape=oshape,
    )(gath, h, h0, b, w)


def _tc_final(x, s, w_en, b_en, batch2d, wf1, bf1, wf2, bf2):
    return pl.pallas_call(
        _final_body,
        in_specs=[
            _whole((N, DF)),
            pl.BlockSpec((2, N, HH), lambda *_: (0, 0, 0)),
            _whole((DF + H, H)),
            _whole((1, H)),
            _whole((1, N)),
            _whole((H, H)),
            _whole((1, H)),
            _whole((H, 1)),
            _whole((1, 1)),
        ],
        out_specs=_whole((G, 1)),
        out_shape=jax.ShapeDtypeStruct((G, 1), _f32),
    )(x, s, w_en, b_en, batch2d, wf1, bf1, wf2, bf2)


MESH = plsc.VectorSubcoreMesh(core_axis_name="c", subcore_axis_name="s")
CHK = 128
EPT = E // 16
NFULL = EPT // CHK
TAIL = EPT - NFULL * CHK
ZR = 624
ZTAIL = N - 16 * ZR
EPW = E // 32
NF32 = EPW // CHK
TAIL32 = EPW - NF32 * CHK


def _scatter_phase(h2, col, A, c, base, dats, idxs, sems_i, sems_d, idxt,
                   gbuft):
    for k in range(2):
        e0 = base + k * CHK
        pltpu.async_copy(col.at[pl.ds(e0, CHK)], idxs[k], sems_i[k])
        pltpu.async_copy(h2.at[c, pl.ds(e0, CHK)], dats[k], sems_d[k])

    def body(j, carry):
        for k in range(2):
            e0 = base + (2 * j + k) * CHK
            pltpu.make_async_copy(col.at[pl.ds(e0, CHK)], idxs[k],
                                  sems_i[k]).wait()
            pltpu.make_async_copy(h2.at[c, pl.ds(e0, CHK)], dats[k],
                                  sems_d[k]).wait()
            pltpu.sync_copy(dats[k], A.at[idxs[k]], add=True)
            en = e0 + 2 * CHK

            @pl.when(2 * j + k + 2 < NFULL)
            def _():
                pltpu.async_copy(col.at[pl.ds(en, CHK)], idxs[k], sems_i[k])
                pltpu.async_copy(h2.at[c, pl.ds(en, CHK)], dats[k], sems_d[k])

        return carry

    lax.fori_loop(0, NFULL // 2, body, 0)
    e0 = base + NFULL * CHK
    pltpu.sync_copy(col.at[pl.ds(e0, TAIL)], idxt)
    pltpu.sync_copy(h2.at[c, pl.ds(e0, TAIL)], gbuft)
    pltpu.sync_copy(gbuft, A.at[idxt], add=True)


def _zero_A(zeros, A, s):
    pltpu.sync_copy(zeros.at[pl.ds(0, ZR)], A.at[pl.ds(s * ZR, ZR)])

    @pl.when(s == 15)
    def _():
        pltpu.sync_copy(zeros.at[pl.ds(0, ZTAIL)], A.at[pl.ds(16 * ZR, ZTAIL)])


def _sc_scatter_gather_body(h2, col, rowi, zeros, gath, A, idx0, idx1, dat0,
                            dat1, idxt, gbuft, si0, si1, sd0, sd1, st0, st1):
    c = lax.axis_index("c")
    s = lax.axis_index("s")
    base = s * EPT
    idxs = (idx0, idx1)
    dats = (dat0, dat1)
    sems_i = (si0, si1)
    sems_d = (sd0, sd1)
    sems_st = (st0, st1)
    _zero_A(zeros, A, s)
    plsc.subcore_barrier()
    _scatter_phase(h2, col, A, c, base, dats, idxs, sems_i, sems_d, idxt,
                   gbuft)
    plsc.subcore_barrier()

    for k in range(2):
        pltpu.async_copy(rowi.at[pl.ds(base + k * CHK, CHK)], idxs[k],
                         sems_i[k])

    def body(j, carry):
        for k in range(2):
            ci = 2 * j + k
            e0 = base + ci * CHK
            pltpu.make_async_copy(rowi.at[pl.ds(e0, CHK)], idxs[k],
                                  sems_i[k]).wait()

            @pl.when(j > 0)
            def _():
                pltpu.make_async_copy(
                    dats[k], gath.at[c, pl.ds(e0 - 2 * CHK, CHK)],
                    sems_st[k]).wait()

            pltpu.sync_copy(A.at[idxs[k]], dats[k])
            pltpu.async_copy(dats[k], gath.at[c, pl.ds(e0, CHK)], sems_st[k])

            @pl.when(ci + 2 < NFULL)
            def _():
                pltpu.async_copy(rowi.at[pl.ds(e0 + 2 * CHK, CHK)], idxs[k],
                                 sems_i[k])

        return carry

    lax.fori_loop(0, NFULL // 2, body, 0)
    for k in range(2):
        ci = NFULL - 2 + k
        pltpu.make_async_copy(dats[k], gath.at[c, pl.ds(base + ci * CHK, CHK)],
                              sems_st[k]).wait()
    e0 = base + NFULL * CHK
    pltpu.sync_copy(rowi.at[pl.ds(e0, TAIL)], idxt)
    pltpu.sync_copy(A.at[idxt], gbuft)
    pltpu.sync_copy(gbuft, gath.at[c, pl.ds(e0, TAIL)])


def _sc_scatter_out_body(h2, col, zeros, s_out, A, idx0, idx1, dat0, dat1,
                         idxt, gbuft, si0, si1, sd0, sd1):
    c = lax.axis_index("c")
    s = lax.axis_index("s")
    base = s * EPT
    _zero_A(zeros, A, s)
    plsc.subcore_barrier()
    _scatter_phase(h2, col, A, c, base, (dat0, dat1), (idx0, idx1),
                   (si0, si1), (sd0, sd1), idxt, gbuft)
    plsc.subcore_barrier()
    pltpu.sync_copy(A.at[pl.ds(s * ZR, ZR)], s_out.at[c, pl.ds(s * ZR, ZR)])

    @pl.when(s == 15)
    def _():
        pltpu.sync_copy(A.at[pl.ds(16 * ZR, ZTAIL)],
                        s_out.at[c, pl.ds(16 * ZR, ZTAIL)])


def _sc_gather_hbm_body(table, rowi, out, rowall, *rest, width, ring):
    dats = rest[:ring]
    idxt, gbuft, semr = rest[ring:ring + 3]
    sems_g = rest[ring + 3:ring + 3 + ring]
    sems_s = rest[ring + 3 + ring:]
    c = lax.axis_index("c")
    s = lax.axis_index("s")
    wid = s * 2 + c
    base = wid * EPW
    pltpu.sync_copy(rowi.at[pl.ds(base, EPW)], rowall)
    for k in range(ring):
        pltpu.async_copy(table.at[rowall.at[pl.ds(k * CHK, CHK)]], dats[k],
                         sems_g[k])

    nr = NF32 // ring

    def body(j, carry):
        for k in range(ring):
            ci = ring * j + k
            pltpu.make_async_copy(
                table.at[rowall.at[pl.ds(ci * CHK, CHK)]], dats[k],
                sems_g[k]).wait()
            pltpu.async_copy(dats[k], out.at[pl.ds(base + ci * CHK, CHK)],
                             sems_s[k])
        for k in range(ring):
            ci = ring * j + k
            pltpu.make_async_copy(dats[k],
                                  out.at[pl.ds(base + ci * CHK, CHK)],
                                  sems_s[k]).wait()

            @pl.when(j < nr - 1)
            def _():
                pltpu.async_copy(
                    table.at[rowall.at[pl.ds((ci + ring) * CHK, CHK)]],
                    dats[k], sems_g[k])

        return carry

    lax.fori_loop(0, nr, body, 0)
    e0 = base + NF32 * CHK
    pltpu.sync_copy(rowi.at[pl.ds(e0, TAIL32)], idxt)
    pltpu.async_copy(table.at[idxt], gbuft, semr).wait()
    pltpu.sync_copy(gbuft, out.at[pl.ds(e0, TAIL32)])


def _sc_gather_hbm(table, rowi, width, ring):
    dma = pltpu.SemaphoreType.DMA
    return pl.kernel(
        functools.partial(_sc_gather_hbm_body, width=width, ring=ring),
        mesh=MESH,
        out_type=jax.ShapeDtypeStruct((E, width), _f32),
        scratch_types=[pltpu.VMEM((EPW,), _i32)]
        + [pltpu.VMEM((CHK, width), _f32)] * ring
        + [pltpu.VMEM((TAIL32,), _i32), pltpu.VMEM((TAIL32, width), _f32)]
        + [dma] * (1 + 2 * ring),
    )(table, rowi)


def _sc_scatter_out(h2, col, zeros):
    dma = pltpu.SemaphoreType.DMA
    return pl.kernel(
        _sc_scatter_out_body,
        mesh=MESH,
        out_type=jax.ShapeDtypeStruct((2, N, HH), _f32),
        scratch_types=[
            pltpu.VMEM_SHARED((N, HH), _f32),
            pltpu.VMEM((CHK,), _i32),
            pltpu.VMEM((CHK,), _i32),
            pltpu.VMEM((CHK, HH), _f32),
            pltpu.VMEM((CHK, HH), _f32),
            pltpu.VMEM((TAIL,), _i32),
            pltpu.VMEM((TAIL, HH), _f32),
        ] + [dma] * 4,
    )(h2, col, zeros)


def kernel(x, edge_attr, W_ei, b_ei, W_conv, b_conv, W_en, b_en, W_f1, b_f1,
           W_f2, b_f2, edge_index, batch, atom_origin_type):
    row = edge_index[0].astype(jnp.int32)
    col = edge_index[1].astype(jnp.int32)
    b_ei2 = b_ei.reshape(1, H)
    b_en2 = b_en.reshape(1, H)
    bf1 = b_f1.reshape(1, H)
    bf2 = b_f2.reshape(1, 1)
    batch2d = batch.astype(jnp.int32).reshape(1, N)
    zeros = jnp.zeros((ZR, HH), _f32)

    xr = _sc_gather_hbm(x, row, DF, 6)
    h = _tc_init(xr, edge_attr, W_ei, b_ei2)
    h0 = h
    for l in range(DEPTH):
        a = jax.ops.segment_sum(h, col, num_segments=N)
        gath = _sc_gather_hbm(a, row, H, 3)
        h = _tc_layer(gath, h, h0, b_conv[l].reshape(1, H), W_conv[l],
                      split=(l == DEPTH - 1))
    s = _sc_scatter_out(h, col, zeros)
    out = _tc_final(x, s, W_en, b_en2, batch2d, W_f1, bf1, W_f2, bf2)
    return out.reshape(G)
